# baseline (speedup 1.0000x reference)
import functools

import jax
import jax.numpy as jnp
from jax import lax
from jax.experimental import pallas as pl
from jax.experimental.pallas import tpu as pltpu
from jax.experimental.pallas import tpu_sc as plsc

VOCAB = 1000
D_MODEL = 128
BATCH = 4096
HIST = 200
TOK = BATCH * HIST

NC = 2
NS = 16
NW = NC * NS
TPW = TOK // NW
CHUNK = 64
NCHUNK = TPW // CHUNK


def _mm_body(e_ref, w_ref, b_ref, m_ref):
    m_ref[...] = lax.dot_general(
        e_ref[...], w_ref[...], (((1,), (1,)), ((), ())),
        preferred_element_type=jnp.float32,
    ) + b_ref[...]


def _fused_table(embed_table, proj_w, proj_b):
    return pl.pallas_call(
        _mm_body,
        out_shape=jax.ShapeDtypeStruct((VOCAB, VOCAB), jnp.float32),
    )(embed_table, proj_w, proj_b.reshape(1, VOCAB))


def _gather_body(m_hbm, emb_hbm, ids_hbm, logits_hbm, hidden_hbm,
                 idx_v, mrow_v, erow_v, sem_m, sem_e):
    wid = lax.axis_index("s") * NC + lax.axis_index("c")
    base = wid * TPW
    pltpu.sync_copy(ids_hbm.at[pl.ds(base, TPW)], idx_v)

    def body(i, carry):
        off = i * CHUNK
        idx_chunk = idx_v.at[pl.ds(off, CHUNK)]
        cp_m = pltpu.async_copy(m_hbm.at[idx_chunk], mrow_v, sem_m)
        cp_e = pltpu.async_copy(emb_hbm.at[idx_chunk], erow_v, sem_e)
        cp_m.wait()
        cp_e.wait()
        pltpu.sync_copy(mrow_v, logits_hbm.at[pl.ds(base + off, CHUNK)])
        pltpu.sync_copy(erow_v, hidden_hbm.at[pl.ds(base + off, CHUNK)])
        return carry

    lax.fori_loop(0, NCHUNK, body, 0)


_gather = functools.partial(
    pl.kernel,
    out_type=[
        jax.ShapeDtypeStruct((TOK, VOCAB), jnp.float32),
        jax.ShapeDtypeStruct((TOK, D_MODEL), jnp.float32),
    ],
    mesh=plsc.VectorSubcoreMesh(core_axis_name="c", subcore_axis_name="s"),
    scratch_types=[
        pltpu.VMEM((TPW,), jnp.int32),
        pltpu.VMEM((CHUNK, VOCAB), jnp.float32),
        pltpu.VMEM((CHUNK, D_MODEL), jnp.float32),
        pltpu.SemaphoreType.DMA,
        pltpu.SemaphoreType.DMA,
    ],
    compiler_params=pltpu.CompilerParams(use_tc_tiling_on_sc=False),
)(_gather_body)


def kernel(input_ids, embed_table, proj_w, proj_b):
    m = _fused_table(embed_table, proj_w, proj_b)
    ids = input_ids.reshape(TOK).astype(jnp.int32)
    logits_flat, hidden_flat = _gather(m, embed_table, ids)
    return (logits_flat.reshape(BATCH, HIST, VOCAB),
            hidden_flat.reshape(BATCH, HIST, D_MODEL))


# speedup vs baseline: 1.1378x; 1.1378x over previous
import functools

import jax
import jax.numpy as jnp
from jax import lax
from jax.experimental import pallas as pl
from jax.experimental.pallas import tpu as pltpu
from jax.experimental.pallas import tpu_sc as plsc

VOCAB = 1000
D_MODEL = 128
BATCH = 4096
HIST = 200
TOK = BATCH * HIST

NC = 2
NS = 16
NW = NC * NS
TPW = TOK // NW
CHUNK = 16
NCHUNK = TPW // CHUNK


def _mm_body(e_ref, w_ref, b_ref, m_ref):
    m_ref[...] = lax.dot_general(
        e_ref[...], w_ref[...], (((1,), (1,)), ((), ())),
        preferred_element_type=jnp.float32,
    ) + b_ref[...]


def _fused_table(embed_table, proj_w, proj_b):
    return pl.pallas_call(
        _mm_body,
        out_shape=jax.ShapeDtypeStruct((VOCAB, VOCAB), jnp.float32),
    )(embed_table, proj_w, proj_b.reshape(1, VOCAB))


def _gather_body(m_hbm, emb_hbm, ids_hbm, logits_hbm, hidden_hbm,
                 m_sh, idx_v,
                 mrow0, erow0, mrow1, erow1,
                 sm0, se0, sm1, se1):
    cid = lax.axis_index("c")
    sid = lax.axis_index("s")
    wid = sid * NC + cid
    base = wid * TPW

    rows = VOCAB // 8

    @pl.when(sid < 8)
    def _stage_m():
        pltpu.sync_copy(m_hbm.at[pl.ds(sid * rows, rows)],
                        m_sh.at[pl.ds(sid * rows, rows)])

    pltpu.sync_copy(ids_hbm.at[pl.ds(base, TPW)], idx_v)
    plsc.subcore_barrier()

    mrow = (mrow0, mrow1)
    erow = (erow0, erow1)
    sm = (sm0, sm1)
    se = (se0, se1)

    def fire(c, b):
        idx_chunk = idx_v.at[pl.ds(c * CHUNK, CHUNK)]
        pltpu.async_copy(m_sh.at[idx_chunk], mrow[b], sm[b])
        pltpu.async_copy(emb_hbm.at[idx_chunk], erow[b], se[b])

    def drain(c, b):
        idx_chunk = idx_v.at[pl.ds(c * CHUNK, CHUNK)]
        pltpu.make_async_copy(m_sh.at[idx_chunk], mrow[b], sm[b]).wait()
        pltpu.make_async_copy(emb_hbm.at[idx_chunk], erow[b], se[b]).wait()

    fire(0, 0)
    fire(1, 1)

    def body(g, carry):
        for b in range(2):
            c = g * 2 + b
            drain(c, b)
            pltpu.sync_copy(mrow[b], logits_hbm.at[pl.ds(base + c * CHUNK, CHUNK)])
            pltpu.sync_copy(erow[b], hidden_hbm.at[pl.ds(base + c * CHUNK, CHUNK)])

            @pl.when(c + 2 < NCHUNK)
            def _next():
                fire(c + 2, b)
        return carry

    lax.fori_loop(0, NCHUNK // 2, body, 0)


_gather = functools.partial(
    pl.kernel,
    out_type=[
        jax.ShapeDtypeStruct((TOK, VOCAB), jnp.float32),
        jax.ShapeDtypeStruct((TOK, D_MODEL), jnp.float32),
    ],
    mesh=plsc.VectorSubcoreMesh(core_axis_name="c", subcore_axis_name="s"),
    scratch_types=[
        pltpu.VMEM_SHARED((VOCAB, VOCAB), jnp.float32),
        pltpu.VMEM((TPW,), jnp.int32),
        pltpu.VMEM((CHUNK, VOCAB), jnp.float32),
        pltpu.VMEM((CHUNK, D_MODEL), jnp.float32),
        pltpu.VMEM((CHUNK, VOCAB), jnp.float32),
        pltpu.VMEM((CHUNK, D_MODEL), jnp.float32),
        pltpu.SemaphoreType.DMA,
        pltpu.SemaphoreType.DMA,
        pltpu.SemaphoreType.DMA,
        pltpu.SemaphoreType.DMA,
    ],
    compiler_params=pltpu.CompilerParams(use_tc_tiling_on_sc=False),
)(_gather_body)


def kernel(input_ids, embed_table, proj_w, proj_b):
    m = _fused_table(embed_table, proj_w, proj_b)
    ids = input_ids.reshape(TOK).astype(jnp.int32)
    logits_flat, hidden_flat = _gather(m, embed_table, ids)
    return (logits_flat.reshape(BATCH, HIST, VOCAB),
            hidden_flat.reshape(BATCH, HIST, D_MODEL))
